# baseline (speedup 1.0000x reference)
import functools

import jax
import jax.numpy as jnp
from jax import lax
from jax.experimental import pallas as pl
from jax.experimental.pallas import tpu as pltpu
from jax.experimental.pallas import tpu_sc as plsc

B = 16384
GMF_DIM = 64
MLP_DIM = 32


def _make_gather_kernel(b_per_w):
    mesh = plsc.VectorSubcoreMesh(core_axis_name="c", subcore_axis_name="s")
    info = plsc.get_sparse_core_info()
    nc = info.num_cores

    @functools.partial(
        pl.kernel,
        mesh=mesh,
        out_type=[
            jax.ShapeDtypeStruct((B, GMF_DIM), jnp.float32),
            jax.ShapeDtypeStruct((B, GMF_DIM), jnp.float32),
            jax.ShapeDtypeStruct((B, MLP_DIM), jnp.float32),
            jax.ShapeDtypeStruct((B, MLP_DIM), jnp.float32),
        ],
        scratch_types=[
            pltpu.VMEM((b_per_w,), jnp.int32),
            pltpu.VMEM((b_per_w,), jnp.int32),
            pltpu.SemaphoreType.DMA,
            pltpu.SemaphoreType.DMA,
        ],
    )
    def gather_kernel(uidx_hbm, iidx_hbm, gmf_user_hbm, gmf_item_hbm,
                      mlp_user_hbm, mlp_item_hbm,
                      gu_out, gi_out, mu_out, mi_out,
                      uidx_v, iidx_v, sem_g, sem_m):
        wid = lax.axis_index("s") * nc + lax.axis_index("c")
        base = wid * b_per_w
        pltpu.sync_copy(uidx_hbm.at[pl.ds(base, b_per_w)], uidx_v)
        pltpu.sync_copy(iidx_hbm.at[pl.ds(base, b_per_w)], iidx_v)

        lanes = 16

        def fire(g, _):
            uvec = uidx_v[pl.ds(g * lanes, lanes)]
            ivec = iidx_v[pl.ds(g * lanes, lanes)]
            for l in range(lanes):
                u = uvec[l]
                it = ivec[l]
                j = base + g * lanes + l
                pltpu.async_copy(
                    gmf_user_hbm.at[pl.ds(u, 1)], gu_out.at[pl.ds(j, 1)],
                    sem_g)
                pltpu.async_copy(
                    gmf_item_hbm.at[pl.ds(it, 1)], gi_out.at[pl.ds(j, 1)],
                    sem_g)
                pltpu.async_copy(
                    mlp_user_hbm.at[pl.ds(u, 1)], mu_out.at[pl.ds(j, 1)],
                    sem_m)
                pltpu.async_copy(
                    mlp_item_hbm.at[pl.ds(it, 1)], mi_out.at[pl.ds(j, 1)],
                    sem_m)
            return 0

        lax.fori_loop(0, b_per_w // lanes, fire, 0)

        def drain(j, _):
            pltpu.make_async_copy(
                gmf_user_hbm.at[pl.ds(0, 1)], gu_out.at[pl.ds(base, 1)],
                sem_g).wait()
            pltpu.make_async_copy(
                gmf_item_hbm.at[pl.ds(0, 1)], gi_out.at[pl.ds(base, 1)],
                sem_g).wait()
            pltpu.make_async_copy(
                mlp_user_hbm.at[pl.ds(0, 1)], mu_out.at[pl.ds(base, 1)],
                sem_m).wait()
            pltpu.make_async_copy(
                mlp_item_hbm.at[pl.ds(0, 1)], mi_out.at[pl.ds(base, 1)],
                sem_m).wait()
            return 0

        lax.fori_loop(0, b_per_w, drain, 0)

    return gather_kernel


def _dense_body(gu, gi, mu, mi, w1a, w1b, b1, w2, b2, w3, b3, wfg, wfh, bf,
                out):
    h = jnp.dot(mu[:], w1a[:], preferred_element_type=jnp.float32)
    h = h + jnp.dot(mi[:], w1b[:], preferred_element_type=jnp.float32)
    h = jnp.maximum(h + b1[:], 0.0)
    h = jnp.maximum(
        jnp.dot(h, w2[:], preferred_element_type=jnp.float32) + b2[:], 0.0)
    h = jnp.maximum(
        jnp.dot(h, w3[:], preferred_element_type=jnp.float32) + b3[:], 0.0)
    g = gu[:] * gi[:]
    s = jnp.dot(g, wfg[:], preferred_element_type=jnp.float32)
    s = s + jnp.dot(h, wfh[:], preferred_element_type=jnp.float32)
    out[:] = jax.nn.sigmoid(s + bf[:])


def kernel(user_indices, item_indices, gmf_user, gmf_item, mlp_user,
           mlp_item, W1, b1, W2, b2, W3, b3, Wf, bf):
    user_indices = user_indices.astype(jnp.int32)
    item_indices = item_indices.astype(jnp.int32)

    nw = 32
    b_per_w = B // nw
    gu, gi, mu, mi = _make_gather_kernel(b_per_w)(
        user_indices, item_indices, gmf_user, gmf_item, mlp_user, mlp_item)

    blk = 2048
    grid = B // blk
    w1a = W1[:MLP_DIM]
    w1b = W1[MLP_DIM:]
    wfg = Wf[:GMF_DIM]
    wfh = Wf[GMF_DIM:]
    rep = lambda shape: pl.BlockSpec(shape, lambda i: (0, 0))
    out = pl.pallas_call(
        _dense_body,
        grid=(grid,),
        in_specs=[
            pl.BlockSpec((blk, GMF_DIM), lambda i: (i, 0)),
            pl.BlockSpec((blk, GMF_DIM), lambda i: (i, 0)),
            pl.BlockSpec((blk, MLP_DIM), lambda i: (i, 0)),
            pl.BlockSpec((blk, MLP_DIM), lambda i: (i, 0)),
            rep((MLP_DIM, 128)),
            rep((MLP_DIM, 128)),
            rep((1, 128)),
            rep((128, 64)),
            rep((1, 64)),
            rep((64, 32)),
            rep((1, 32)),
            rep((GMF_DIM, 1)),
            rep((32, 1)),
            rep((1, 1)),
        ],
        out_specs=pl.BlockSpec((blk, 1), lambda i: (i, 0)),
        out_shape=jax.ShapeDtypeStruct((B, 1), jnp.float32),
    )(gu, gi, mu, mi, w1a, w1b, b1.reshape(1, -1), W2, b2.reshape(1, -1),
      W3, b3.reshape(1, -1), wfg, wfh, bf.reshape(1, 1))
    return out[:, 0]


# speedup vs baseline: 5.8660x; 5.8660x over previous
import functools

import jax
import jax.numpy as jnp
from jax import lax
from jax.experimental import pallas as pl
from jax.experimental.pallas import tpu as pltpu
from jax.experimental.pallas import tpu_sc as plsc

B = 16384
GMF_DIM = 64
MLP_DIM = 32
CH = 128
LN = 16


def _make_gather_kernel(b_per_w):
    mesh = plsc.VectorSubcoreMesh(core_axis_name="c", subcore_axis_name="s")
    info = plsc.get_sparse_core_info()
    nc = info.num_cores

    @functools.partial(
        pl.kernel,
        mesh=mesh,
        out_type=[
            jax.ShapeDtypeStruct((B, GMF_DIM), jnp.float32),
            jax.ShapeDtypeStruct((B, GMF_DIM), jnp.float32),
            jax.ShapeDtypeStruct((B, MLP_DIM), jnp.float32),
            jax.ShapeDtypeStruct((B, MLP_DIM), jnp.float32),
        ],
        scratch_types=[
            pltpu.VMEM((b_per_w,), jnp.int32),
            pltpu.VMEM((b_per_w,), jnp.int32),
            pltpu.VMEM((CH, GMF_DIM), jnp.float32),
            pltpu.VMEM((CH, GMF_DIM), jnp.float32),
            pltpu.VMEM((CH, MLP_DIM), jnp.float32),
            pltpu.VMEM((CH, MLP_DIM), jnp.float32),
            pltpu.SemaphoreType.DMA,
            pltpu.SemaphoreType.DMA,
        ],
    )
    def gather_kernel(uidx_hbm, iidx_hbm, gmf_user_hbm, gmf_item_hbm,
                      mlp_user_hbm, mlp_item_hbm,
                      gu_out, gi_out, mu_out, mi_out,
                      uidx_v, iidx_v, gu_v, gi_v, mu_v, mi_v, sem_g, sem_m):
        wid = lax.axis_index("s") * nc + lax.axis_index("c")
        base = wid * b_per_w
        pltpu.sync_copy(uidx_hbm.at[pl.ds(base, b_per_w)], uidx_v)
        pltpu.sync_copy(iidx_hbm.at[pl.ds(base, b_per_w)], iidx_v)

        def chunk(c, _):
            def fire(g, _):
                uvec = uidx_v[pl.ds(c * CH + g * LN, LN)]
                ivec = iidx_v[pl.ds(c * CH + g * LN, LN)]
                for l in range(LN):
                    u = uvec[l]
                    it = ivec[l]
                    j = g * LN + l
                    pltpu.async_copy(
                        gmf_user_hbm.at[pl.ds(u, 1)], gu_v.at[pl.ds(j, 1)],
                        sem_g)
                    pltpu.async_copy(
                        gmf_item_hbm.at[pl.ds(it, 1)], gi_v.at[pl.ds(j, 1)],
                        sem_g)
                    pltpu.async_copy(
                        mlp_user_hbm.at[pl.ds(u, 1)], mu_v.at[pl.ds(j, 1)],
                        sem_m)
                    pltpu.async_copy(
                        mlp_item_hbm.at[pl.ds(it, 1)], mi_v.at[pl.ds(j, 1)],
                        sem_m)
                return 0

            lax.fori_loop(0, CH // LN, fire, 0)

            def drain(j, _):
                pltpu.make_async_copy(
                    gmf_user_hbm.at[pl.ds(0, 1)], gu_v.at[pl.ds(j, 1)],
                    sem_g).wait()
                pltpu.make_async_copy(
                    gmf_item_hbm.at[pl.ds(0, 1)], gi_v.at[pl.ds(j, 1)],
                    sem_g).wait()
                pltpu.make_async_copy(
                    mlp_user_hbm.at[pl.ds(0, 1)], mu_v.at[pl.ds(j, 1)],
                    sem_m).wait()
                pltpu.make_async_copy(
                    mlp_item_hbm.at[pl.ds(0, 1)], mi_v.at[pl.ds(j, 1)],
                    sem_m).wait()
                return 0

            lax.fori_loop(0, CH, drain, 0)

            pltpu.sync_copy(gu_v, gu_out.at[pl.ds(base + c * CH, CH)])
            pltpu.sync_copy(gi_v, gi_out.at[pl.ds(base + c * CH, CH)])
            pltpu.sync_copy(mu_v, mu_out.at[pl.ds(base + c * CH, CH)])
            pltpu.sync_copy(mi_v, mi_out.at[pl.ds(base + c * CH, CH)])
            return 0

        lax.fori_loop(0, b_per_w // CH, chunk, 0)

    return gather_kernel


def _dense_body(gu, gi, mu, mi, w1a, w1b, b1, w2, b2, w3, b3, wfg, wfh, bf,
                out):
    h = jnp.dot(mu[:], w1a[:], preferred_element_type=jnp.float32)
    h = h + jnp.dot(mi[:], w1b[:], preferred_element_type=jnp.float32)
    h = jnp.maximum(h + b1[:], 0.0)
    h = jnp.maximum(
        jnp.dot(h, w2[:], preferred_element_type=jnp.float32) + b2[:], 0.0)
    h = jnp.maximum(
        jnp.dot(h, w3[:], preferred_element_type=jnp.float32) + b3[:], 0.0)
    g = gu[:] * gi[:]
    s = jnp.dot(g, wfg[:], preferred_element_type=jnp.float32)
    s = s + jnp.dot(h, wfh[:], preferred_element_type=jnp.float32)
    out[:] = jax.nn.sigmoid(s + bf[:])


def kernel(user_indices, item_indices, gmf_user, gmf_item, mlp_user,
           mlp_item, W1, b1, W2, b2, W3, b3, Wf, bf):
    user_indices = user_indices.astype(jnp.int32)
    item_indices = item_indices.astype(jnp.int32)

    nw = 32
    b_per_w = B // nw
    gu, gi, mu, mi = _make_gather_kernel(b_per_w)(
        user_indices, item_indices, gmf_user, gmf_item, mlp_user, mlp_item)

    blk = 2048
    grid = B // blk
    w1a = W1[:MLP_DIM]
    w1b = W1[MLP_DIM:]
    wfg = Wf[:GMF_DIM]
    wfh = Wf[GMF_DIM:]
    rep = lambda shape: pl.BlockSpec(shape, lambda i: (0, 0))
    out = pl.pallas_call(
        _dense_body,
        grid=(grid,),
        in_specs=[
            pl.BlockSpec((blk, GMF_DIM), lambda i: (i, 0)),
            pl.BlockSpec((blk, GMF_DIM), lambda i: (i, 0)),
            pl.BlockSpec((blk, MLP_DIM), lambda i: (i, 0)),
            pl.BlockSpec((blk, MLP_DIM), lambda i: (i, 0)),
            rep((MLP_DIM, 128)),
            rep((MLP_DIM, 128)),
            rep((1, 128)),
            rep((128, 64)),
            rep((1, 64)),
            rep((64, 32)),
            rep((1, 32)),
            rep((GMF_DIM, 1)),
            rep((32, 1)),
            rep((1, 1)),
        ],
        out_specs=pl.BlockSpec((blk, 1), lambda i: (i, 0)),
        out_shape=jax.ShapeDtypeStruct((B, 1), jnp.float32),
    )(gu, gi, mu, mi, w1a, w1b, b1.reshape(1, -1), W2, b2.reshape(1, -1),
      W3, b3.reshape(1, -1), wfg, wfh, bf.reshape(1, 1))
    return out[:, 0]
